# baseline (speedup 1.0000x reference)
import functools
import jax
import jax.numpy as jnp
from jax import lax
from jax.experimental import pallas as pl
from jax.experimental.pallas import tpu as pltpu
from jax.experimental.pallas import tpu_sc as plsc

NT = 8
BB = 8
NC = 2
NS = 16
LANES = 16


def _vals_sc_kernel(ids_hbm, table_hbm, vals_hbm, ids_v, table_v, vals_v):
    B_per_w = 4
    T = 128
    wid = lax.axis_index("s") * NC + lax.axis_index("c")
    base = wid * (B_per_w * T)
    pltpu.sync_copy(ids_hbm.at[pl.ds(base, B_per_w * T)], ids_v)
    pltpu.sync_copy(table_hbm, table_v)

    for b in range(B_per_w):
        jchunks = [
            jnp.clip(ids_v[pl.ds(b * T + jc * LANES, LANES)], 0, NT - 1)
            for jc in range(T // LANES)
        ]

        def row_body(i, _):
            iv = plsc.load_gather(
                ids_v, [jnp.full((LANES,), b * T + i, jnp.int32)])
            riv = jnp.clip(iv, 0, NT - 1) * NT
            for jc in range(T // LANES):
                g = plsc.load_gather(table_v, [riv + jchunks[jc]])
                vals_v[pl.ds(b * T * T + i * T + jc * LANES, LANES)] = g
            return 0

        lax.fori_loop(0, T, row_body, 0)

    pltpu.sync_copy(vals_v, vals_hbm.at[pl.ds(base * T, B_per_w * T * T)])


def _sc_vals(tokens_id, binary_table):
    B, T = tokens_id.shape
    mesh = plsc.VectorSubcoreMesh(core_axis_name="c", subcore_axis_name="s")
    k = functools.partial(
        pl.kernel,
        mesh=mesh,
        out_type=jax.ShapeDtypeStruct((B * T * T,), jnp.float32),
        scratch_types=[
            pltpu.VMEM((B * T // (NC * NS),), jnp.int32),
            pltpu.VMEM((NT * NT,), jnp.float32),
            pltpu.VMEM((B * T * T // (NC * NS),), jnp.float32),
        ],
        compiler_params=pltpu.CompilerParams(needs_layout_passes=False),
    )(_vals_sc_kernel)
    return k(tokens_id.reshape(B * T), binary_table.reshape(NT * NT))


def _bias_tc_kernel(vals_ref, w_ref, b_ref, gate_ref, out_ref):
    vals = vals_ref[:, :, :]
    tg = jnp.tanh(gate_ref[0, 0])
    scale = tg * w_ref[0, :]
    offset = tg * b_ref[0, :]
    out_ref[:, :, :, :] = (vals[:, None, :, :] * scale[None, :, None, None]
                           + offset[None, :, None, None])


def kernel(tokens_id, W, b, gate, binary_table):
    B, T = tokens_id.shape
    H = W.shape[0]
    vals = _sc_vals(tokens_id, binary_table).reshape(B, T, T)
    w2 = W.reshape(1, H)
    b2 = b.reshape(1, H)
    gate2 = gate.reshape(1, 1)
    return pl.pallas_call(
        _bias_tc_kernel,
        grid=(B // BB,),
        in_specs=[
            pl.BlockSpec((BB, T, T), lambda i: (i, 0, 0)),
            pl.BlockSpec((1, H), lambda i: (0, 0)),
            pl.BlockSpec((1, H), lambda i: (0, 0)),
            pl.BlockSpec((1, 1), lambda i: (0, 0)),
        ],
        out_specs=pl.BlockSpec((BB, H, T, T), lambda i: (i, 0, 0, 0)),
        out_shape=jax.ShapeDtypeStruct((B, H, T, T), jnp.float32),
    )(vals, w2, b2, gate2)


# speedup vs baseline: 2.0868x; 2.0868x over previous
import jax
import jax.numpy as jnp
from jax.experimental import pallas as pl

NT = 8


BB = 8


def _bias_kernel(ids_ref, w_ref, b_ref, gate_ref, table_ref, out_ref):
    ids = jnp.clip(ids_ref[:, 0, :], 0, NT - 1)
    bb, T = ids.shape
    iota = jax.lax.broadcasted_iota(jnp.int32, (bb, T, NT), 2)
    onehot = (ids[:, :, None] == iota).astype(jnp.float32)
    rows = jax.lax.dot_general(
        onehot, table_ref[:, :],
        dimension_numbers=(((2,), (0,)), ((), ())),
        preferred_element_type=jnp.float32)
    vals = jax.lax.dot_general(
        rows, onehot,
        dimension_numbers=(((2,), (2,)), ((0,), (0,))),
        preferred_element_type=jnp.float32)
    tg = jnp.tanh(gate_ref[0, 0])
    scale = tg * w_ref[0, :]
    offset = tg * b_ref[0, :]
    out_ref[:, :, :, :] = (vals[:, None, :, :] * scale[None, :, None, None]
                           + offset[None, :, None, None])


def kernel(tokens_id, W, b, gate, binary_table):
    B, T = tokens_id.shape
    H = W.shape[0]
    ids3 = tokens_id.reshape(B, 1, T)
    w2 = W.reshape(1, H)
    b2 = b.reshape(1, H)
    gate2 = gate.reshape(1, 1)
    return pl.pallas_call(
        _bias_kernel,
        grid=(B // BB,),
        in_specs=[
            pl.BlockSpec((BB, 1, T), lambda i: (i, 0, 0)),
            pl.BlockSpec((1, H), lambda i: (0, 0)),
            pl.BlockSpec((1, H), lambda i: (0, 0)),
            pl.BlockSpec((1, 1), lambda i: (0, 0)),
            pl.BlockSpec((NT, NT), lambda i: (0, 0)),
        ],
        out_specs=pl.BlockSpec((BB, H, T, T), lambda i: (i, 0, 0, 0)),
        out_shape=jax.ShapeDtypeStruct((B, H, T, T), jnp.float32),
    )(ids3, w2, b2, gate2, binary_table)
